# baseline (speedup 1.0000x reference)
import functools

import jax
import jax.numpy as jnp
from jax import lax
from jax.experimental import pallas as pl
from jax.experimental.pallas import tpu as pltpu
from jax.experimental.pallas import tpu_sc as plsc

NUM_USERS = 1000000
DIM = 32
BATCH = 16384
_SUB = 8
_L = 16
_WAVE = 16

_info = plsc.get_sparse_core_info()
_NC, _NS = _info.num_cores, _info.num_subcores
_NW = _NC * _NS
_B_PER_W = BATCH // _NW
_NVEC = _B_PER_W // _L


_mesh = plsc.VectorSubcoreMesh(core_axis_name="c", subcore_axis_name="s")


@functools.partial(
    pl.kernel,
    mesh=_mesh,
    out_type=jax.ShapeDtypeStruct((DIM, BATCH), jnp.float32),
    scratch_types=[
        pltpu.VMEM((_B_PER_W,), jnp.int32),
        pltpu.VMEM((_B_PER_W,), jnp.int32),
        pltpu.VMEM((_WAVE, _B_PER_W, _SUB), jnp.float32),
        pltpu.VMEM((DIM, _B_PER_W), jnp.float32),
        pltpu.SemaphoreType.DMA,
    ],
    compiler_params=pltpu.CompilerParams(
        use_tc_tiling_on_sc=False, needs_layout_passes=False
    ),
)
def _gather_kernel(idx_hbm, table_hbm, out_hbm, idx_v, idxq_v, raw_v, pk_v, sem):
    wid = lax.axis_index("s") * _NC + lax.axis_index("c")
    base = wid * _B_PER_W
    pltpu.sync_copy(idx_hbm.at[pl.ds(base, _B_PER_W)], idx_v)
    for k in range(_NVEC):
        idxq_v[pl.ds(k * _L, _L)] = lax.shift_right_logical(
            idx_v[pl.ds(k * _L, _L)], 3
        )
    iota = lax.iota(jnp.int32, _L)

    def gather_col(c):
        src = table_hbm.at[pl.ds(c * (NUM_USERS // _SUB), NUM_USERS // _SUB)]
        return pltpu.async_copy(src.at[idxq_v], raw_v.at[c % _WAVE], sem)

    def select_col(c):
        def body(k, _, c=c):
            row16 = k * _L + iota
            rem16 = lax.bitwise_and(idx_v[pl.ds(k * _L, _L)], _SUB - 1)
            v = plsc.load_gather(raw_v.at[c % _WAVE], [row16, rem16])
            pk_v[c, pl.ds(k * _L, _L)] = v
            return 0

        lax.fori_loop(0, _NVEC, body, 0)

    wave_a = [gather_col(c) for c in range(_WAVE)]
    wave_b = []
    for c in range(_WAVE):
        wave_a[c].wait()
        select_col(c)
        wave_b.append(gather_col(_WAVE + c))
    for c in range(_WAVE):
        wave_b[c].wait()
        select_col(_WAVE + c)
    for c in range(DIM):
        pltpu.sync_copy(pk_v.at[c], out_hbm.at[c].at[pl.ds(base, _B_PER_W)])


def kernel(x, table):
    table_t = table.T.reshape(DIM * NUM_USERS // _SUB, _SUB)
    out_t = _gather_kernel(x.astype(jnp.int32), table_t)
    return out_t.T


# speedup vs baseline: 22.8472x; 22.8472x over previous
import functools

import jax
import jax.numpy as jnp
from jax import lax
from jax.experimental import pallas as pl
from jax.experimental.pallas import tpu as pltpu
from jax.experimental.pallas import tpu_sc as plsc

NUM_USERS = 1000000
DIM = 32
BATCH = 16384
_W = 128
_L = 16
_RING = 8

_info = plsc.get_sparse_core_info()
_NC, _NS = _info.num_cores, _info.num_subcores
_NW = _NC * _NS
_B_PER_W = BATCH // _NW


_mesh = plsc.VectorSubcoreMesh(core_axis_name="c", subcore_axis_name="s")


@functools.partial(
    pl.kernel,
    mesh=_mesh,
    out_type=jax.ShapeDtypeStruct((DIM, BATCH), jnp.float32),
    scratch_types=[
        pltpu.VMEM((_B_PER_W + _L,), jnp.int32),
        pltpu.VMEM((_RING, DIM, _W), jnp.float32),
        pltpu.VMEM((DIM, _B_PER_W), jnp.float32),
        pltpu.SemaphoreType.DMA,
    ],
    compiler_params=pltpu.CompilerParams(needs_layout_passes=False),
)
def _gather_kernel(idx_hbm, table_t_hbm, out_t_hbm, idx_v, win_v, pk_v, sem):
    wid = lax.axis_index("s") * _NC + lax.axis_index("c")
    base = wid * _B_PER_W
    pltpu.sync_copy(idx_hbm.at[pl.ds(base, _B_PER_W)], idx_v.at[pl.ds(0, _B_PER_W)])
    iota = lax.iota(jnp.int32, _L)

    def fetch(i, slot):
        u = idx_v[pl.ds(i, _L)][0]
        off = pl.multiple_of((u // _W) * _W, _W)
        pltpu.async_copy(
            table_t_hbm.at[:, pl.ds(off, _W)],
            win_v.at[slot],
            sem,
        )

    def drain_one():
        pltpu.make_async_copy(
            table_t_hbm.at[:, pl.ds(0, _W)], win_v.at[0], sem
        ).wait()

    def extract(i, slot):
        u = idx_v[pl.ds(i, _L)][0]
        lane = jnp.full((_L,), lax.rem(u, _W), jnp.int32)
        col = jnp.full((_L,), i, jnp.int32)
        for h in range(DIM // _L):
            v = plsc.load_gather(win_v.at[slot], [h * _L + iota, lane])
            plsc.store_scatter(pk_v, [h * _L + iota, col], v)

    for i in range(_RING):
        fetch(i, i)

    def body(i, _):
        slot = lax.rem(i, _RING)
        drain_one()
        extract(i, slot)
        fetch(i + _RING, slot)
        return 0

    lax.fori_loop(0, _B_PER_W - _RING, body, 0)
    for j in range(_RING):
        i = _B_PER_W - _RING + j
        drain_one()
        extract(i, i % _RING)
    pltpu.sync_copy(pk_v, out_t_hbm.at[:, pl.ds(base, _B_PER_W)])


def kernel(x, table):
    out_t = _gather_kernel(x.astype(jnp.int32), table.T)
    return out_t.T
